# baseline (speedup 1.0000x reference)
import functools

import jax
import jax.numpy as jnp
from jax import lax
from jax.experimental import pallas as pl
from jax.experimental.pallas import tpu as pltpu
from jax.experimental.pallas import tpu_sc as plsc

DIM = 300
DIM_PAD = 384
SEQ = 50
SEQ_PAD = 56
NCHUNK = 2
SLICE_BLK = 16


def _pad_cols_tc(table):
    v = table.shape[0]
    blk = 2000

    def body(in_ref, out_ref):
        out_ref[:, :DIM] = in_ref[...]

    return pl.pallas_call(
        body,
        grid=(v // blk,),
        in_specs=[pl.BlockSpec((blk, DIM), lambda i: (i, 0))],
        out_specs=pl.BlockSpec((blk, DIM_PAD), lambda i: (i, 0)),
        out_shape=jax.ShapeDtypeStruct((v, DIM_PAD), jnp.float32),
    )(table)


def _embed_gather(idx_grp, table, n_batch, num_cores, num_subcores):
    b_per_w = idx_grp.shape[1]

    mesh = plsc.VectorSubcoreMesh(core_axis_name="c", subcore_axis_name="s")

    nbuf = 4

    @functools.partial(
        pl.kernel,
        mesh=mesh,
        out_type=jax.ShapeDtypeStruct((n_batch * SEQ_PAD, DIM_PAD), jnp.float32),
        scratch_types=[
            pltpu.VMEM((b_per_w, SEQ_PAD), jnp.int32),
            [pltpu.VMEM((SEQ_PAD, DIM_PAD), jnp.float32) for _ in range(nbuf)],
            [pltpu.SemaphoreType.DMA for _ in range(nbuf)],
            [pltpu.SemaphoreType.DMA for _ in range(nbuf)],
        ],
    )
    def k(idx_hbm, table_hbm, out_hbm, idx_v, rows, gsems, wsems):
        wid = lax.axis_index("s") * num_cores + lax.axis_index("c")
        base_b = wid * b_per_w
        pltpu.sync_copy(idx_hbm.at[wid], idx_v)

        def g_start(jb, p):
            pltpu.async_copy(table_hbm.at[idx_v.at[jb]], rows[p], gsems[p])

        def wb_start(jb, p):
            pltpu.async_copy(
                rows[p], out_hbm.at[pl.ds((base_b + jb) * SEQ_PAD, SEQ_PAD)],
                wsems[p],
            )

        for p in range(nbuf):
            g_start(p, p)

        @pl.loop(0, b_per_w, step=nbuf)
        def _(jb):
            for p in range(nbuf):
                j = jb + p
                pltpu.make_async_copy(
                    table_hbm.at[idx_v.at[j]], rows[p], gsems[p]
                ).wait()
                wb_start(j, p)

                @pl.when(j + nbuf < b_per_w)
                def _():
                    pltpu.make_async_copy(
                        rows[p],
                        out_hbm.at[pl.ds((base_b + j) * SEQ_PAD, SEQ_PAD)],
                        wsems[p],
                    ).wait()
                    g_start(j + nbuf, p)

        for p in range(nbuf):
            j = b_per_w - nbuf + p
            pltpu.make_async_copy(
                rows[p],
                out_hbm.at[pl.ds((base_b + j) * SEQ_PAD, SEQ_PAD)],
                wsems[p],
            ).wait()

    return k(idx_grp, table)


def _compact_tc(gathered, acc, chunk, nb_chunk):
    b_total = acc.shape[0]
    nblk = nb_chunk // SLICE_BLK
    base = chunk * nblk

    def body(in_ref, _, out_ref):
        blk = in_ref[...].reshape(SLICE_BLK, SEQ_PAD, DIM_PAD)
        out_ref[...] = blk[:, :SEQ, :DIM]

    return pl.pallas_call(
        body,
        grid=(nblk,),
        in_specs=[
            pl.BlockSpec((SLICE_BLK * SEQ_PAD, DIM_PAD), lambda i: (i, 0)),
            pl.BlockSpec(memory_space=pl.ANY),
        ],
        out_specs=pl.BlockSpec((SLICE_BLK, SEQ, DIM), lambda i: (base + i, 0, 0)),
        out_shape=jax.ShapeDtypeStruct((b_total, SEQ, DIM), jnp.float32),
        input_output_aliases={1: 0},
    )(gathered, acc)


def kernel(xc, xw, table):
    del xc
    b, l = xw.shape
    info = plsc.get_sparse_core_info()
    nw = info.num_cores * info.num_subcores
    nb_chunk = b // NCHUNK
    idx = xw.astype(jnp.int32).reshape(NCHUNK, nw, nb_chunk // nw, l)
    idx = jnp.pad(idx, ((0, 0), (0, 0), (0, 0), (0, SEQ_PAD - SEQ)), mode="edge")
    table_p = _pad_cols_tc(table)
    acc = jnp.zeros((b, SEQ, DIM), jnp.float32)
    for c in range(NCHUNK):
        g = _embed_gather(
            idx[c], table_p, nb_chunk, info.num_cores, info.num_subcores
        )
        acc = _compact_tc(g, acc, c, nb_chunk)
    return acc


# speedup vs baseline: 1.1302x; 1.1302x over previous
import functools

import jax
import jax.numpy as jnp
from jax import lax
from jax.experimental import pallas as pl
from jax.experimental.pallas import tpu as pltpu
from jax.experimental.pallas import tpu_sc as plsc

DIM = 300
DIM_PAD = 384
SEQ = 50
SEQ_PAD = 56
NCHUNK = 2
SLICE_BLK = 16


def _pad_cols_tc(table):
    v = table.shape[0]
    blk = 2000

    def body(in_ref, out_ref):
        out_ref[:, :DIM] = in_ref[...]

    return pl.pallas_call(
        body,
        grid=(v // blk,),
        in_specs=[pl.BlockSpec((blk, DIM), lambda i: (i, 0))],
        out_specs=pl.BlockSpec((blk, DIM_PAD), lambda i: (i, 0)),
        out_shape=jax.ShapeDtypeStruct((v, DIM_PAD), jnp.float32),
    )(table)


def _embed_gather(idx_grp, table, n_batch, num_cores, num_subcores):
    b_per_w = idx_grp.shape[1]

    mesh = plsc.VectorSubcoreMesh(core_axis_name="c", subcore_axis_name="s")

    nbuf = 4

    @functools.partial(
        pl.kernel,
        mesh=mesh,
        out_type=jax.ShapeDtypeStruct((n_batch * SEQ_PAD, DIM_PAD), jnp.float32),
        scratch_types=[
            pltpu.VMEM((b_per_w, SEQ_PAD), jnp.int32),
            [pltpu.VMEM((SEQ_PAD, DIM_PAD), jnp.float32) for _ in range(nbuf)],
            [pltpu.SemaphoreType.DMA for _ in range(nbuf)],
            [pltpu.SemaphoreType.DMA for _ in range(nbuf)],
        ],
    )
    def k(idx_hbm, table_hbm, out_hbm, idx_v, rows, gsems, wsems):
        wid = lax.axis_index("s") * num_cores + lax.axis_index("c")
        base_b = wid * b_per_w
        pltpu.sync_copy(idx_hbm.at[wid], idx_v)

        def g_start(jb, p):
            pltpu.async_copy(table_hbm.at[idx_v.at[jb]], rows[p], gsems[p])

        def wb_start(jb, p):
            pltpu.async_copy(
                rows[p], out_hbm.at[pl.ds((base_b + jb) * SEQ_PAD, SEQ_PAD)],
                wsems[p],
            )

        for p in range(nbuf):
            g_start(p, p)

        @pl.loop(0, b_per_w, step=nbuf)
        def _(jb):
            for p in range(nbuf):
                j = jb + p
                pltpu.make_async_copy(
                    table_hbm.at[idx_v.at[j]], rows[p], gsems[p]
                ).wait()
                wb_start(j, p)

                @pl.when(j + nbuf < b_per_w)
                def _():
                    pltpu.make_async_copy(
                        rows[p],
                        out_hbm.at[pl.ds((base_b + j) * SEQ_PAD, SEQ_PAD)],
                        wsems[p],
                    ).wait()
                    g_start(j + nbuf, p)

        for p in range(nbuf):
            j = b_per_w - nbuf + p
            pltpu.make_async_copy(
                rows[p],
                out_hbm.at[pl.ds((base_b + j) * SEQ_PAD, SEQ_PAD)],
                wsems[p],
            ).wait()

    return k(idx_grp, table)


def _compact_tc(gathered, acc, chunk, nb_chunk, b_total):
    nblk = nb_chunk // SLICE_BLK
    base = chunk * nblk

    def body(in_ref, *refs):
        out_ref = refs[-1]
        blk = in_ref[...].reshape(SLICE_BLK, SEQ_PAD, DIM_PAD)
        out_ref[...] = blk[:, :SEQ, :DIM]

    in_specs = [pl.BlockSpec((SLICE_BLK * SEQ_PAD, DIM_PAD), lambda i: (i, 0))]
    operands = [gathered]
    aliases = {}
    if acc is not None:
        in_specs.append(pl.BlockSpec(memory_space=pl.ANY))
        operands.append(acc)
        aliases = {1: 0}

    return pl.pallas_call(
        body,
        grid=(nblk,),
        in_specs=in_specs,
        out_specs=pl.BlockSpec((SLICE_BLK, SEQ, DIM), lambda i: (base + i, 0, 0)),
        out_shape=jax.ShapeDtypeStruct((b_total, SEQ, DIM), jnp.float32),
        input_output_aliases=aliases,
    )(*operands)


def kernel(xc, xw, table):
    del xc
    b, l = xw.shape
    info = plsc.get_sparse_core_info()
    nw = info.num_cores * info.num_subcores
    nb_chunk = b // NCHUNK
    idx = xw.astype(jnp.int32).reshape(NCHUNK, nw, nb_chunk // nw, l)
    idx = jnp.pad(idx, ((0, 0), (0, 0), (0, 0), (0, SEQ_PAD - SEQ)), mode="edge")
    table_p = _pad_cols_tc(table)
    acc = None
    for c in range(NCHUNK):
        g = _embed_gather(
            idx[c], table_p, nb_chunk, info.num_cores, info.num_subcores
        )
        acc = _compact_tc(g, acc, c, nb_chunk, b)
    return acc


# speedup vs baseline: 1.2396x; 1.0968x over previous
import functools

import jax
import jax.numpy as jnp
from jax import lax
from jax.experimental import pallas as pl
from jax.experimental.pallas import tpu as pltpu
from jax.experimental.pallas import tpu_sc as plsc

DIM = 300
DIM_PAD = 384
SEQ = 50
SEQ_PAD = 56


def _pad_cols_tc(table):
    v = table.shape[0]
    blk = 2000

    def body(in_ref, out_ref):
        out_ref[:, :DIM] = in_ref[...]

    return pl.pallas_call(
        body,
        grid=(v // blk,),
        in_specs=[pl.BlockSpec((blk, DIM), lambda i: (i, 0))],
        out_specs=pl.BlockSpec((blk, DIM_PAD), lambda i: (i, 0)),
        out_shape=jax.ShapeDtypeStruct((v, DIM_PAD), jnp.float32),
    )(table)


def _embed_gather(idx_grp, table, n_batch, num_cores, num_subcores):
    b_per_w = idx_grp.shape[1]

    mesh = plsc.VectorSubcoreMesh(core_axis_name="c", subcore_axis_name="s")

    nbuf = 4

    @functools.partial(
        pl.kernel,
        mesh=mesh,
        out_type=jax.ShapeDtypeStruct((n_batch * SEQ_PAD, DIM_PAD), jnp.float32),
        scratch_types=[
            pltpu.VMEM((b_per_w, SEQ_PAD), jnp.int32),
            [pltpu.VMEM((SEQ_PAD, DIM_PAD), jnp.float32) for _ in range(nbuf)],
            [pltpu.SemaphoreType.DMA for _ in range(nbuf)],
            [pltpu.SemaphoreType.DMA for _ in range(nbuf)],
        ],
    )
    def k(idx_hbm, table_hbm, out_hbm, idx_v, rows, gsems, wsems):
        wid = lax.axis_index("s") * num_cores + lax.axis_index("c")
        base_b = wid * b_per_w
        pltpu.sync_copy(idx_hbm.at[wid], idx_v)

        def g_start(jb, p):
            pltpu.async_copy(table_hbm.at[idx_v.at[jb]], rows[p], gsems[p])

        def wb_start(jb, p):
            pltpu.async_copy(
                rows[p], out_hbm.at[pl.ds((base_b + jb) * SEQ_PAD, SEQ_PAD)],
                wsems[p],
            )

        for p in range(nbuf):
            g_start(p, p)

        @pl.loop(0, b_per_w, step=nbuf)
        def _(jb):
            for p in range(nbuf):
                j = jb + p
                pltpu.make_async_copy(
                    table_hbm.at[idx_v.at[j]], rows[p], gsems[p]
                ).wait()
                wb_start(j, p)

                @pl.when(j + nbuf < b_per_w)
                def _():
                    pltpu.make_async_copy(
                        rows[p],
                        out_hbm.at[pl.ds((base_b + j) * SEQ_PAD, SEQ_PAD)],
                        wsems[p],
                    ).wait()
                    g_start(j + nbuf, p)

        for p in range(nbuf):
            j = b_per_w - nbuf + p
            pltpu.make_async_copy(
                rows[p],
                out_hbm.at[pl.ds((base_b + j) * SEQ_PAD, SEQ_PAD)],
                wsems[p],
            ).wait()

    return k(idx_grp, table)


SLICE_BLK = 16


def _compact_half_tc(gathered, b_total, b_half):
    nblk = b_half // SLICE_BLK

    def body(in_ref, out_ref):
        blk = in_ref[...].reshape(SLICE_BLK, SEQ_PAD, DIM_PAD)
        out_ref[...] = blk[:, :SEQ, :DIM]

    return pl.pallas_call(
        body,
        grid=(nblk,),
        in_specs=[pl.BlockSpec((SLICE_BLK * SEQ_PAD, DIM_PAD), lambda i: (i, 0))],
        out_specs=pl.BlockSpec((SLICE_BLK, SEQ, DIM), lambda i: (i, 0, 0)),
        out_shape=jax.ShapeDtypeStruct((b_total, SEQ, DIM), jnp.float32),
    )(gathered)


def kernel(xc, xw, table):
    del xc
    b, l = xw.shape
    info = plsc.get_sparse_core_info()
    nw = info.num_cores * info.num_subcores
    idx = xw.reshape(nw, b // nw, l).astype(jnp.int32)
    idx = jnp.pad(idx, ((0, 0), (0, 0), (0, SEQ_PAD - SEQ)), mode="edge")
    table_p = _pad_cols_tc(table)
    out = _embed_gather(idx, table_p, b, info.num_cores, info.num_subcores)
    b_half = b // 2
    half0 = _compact_half_tc(out, b, b_half)
    s1 = out.reshape(b, SEQ_PAD, DIM_PAD)[b_half:, :SEQ, :DIM]
    return lax.dynamic_update_slice(half0, s1, (b_half, 0, 0))


# speedup vs baseline: 1.6892x; 1.3627x over previous
import functools

import jax
import jax.numpy as jnp
from jax import lax
from jax.experimental import pallas as pl
from jax.experimental.pallas import tpu as pltpu
from jax.experimental.pallas import tpu_sc as plsc

DIM = 300
DIM_PAD = 384
SEQ = 50
SEQ_PAD = 56


def _pad_cols_tc(table):
    v = table.shape[0]
    blk = 2000

    def body(in_ref, out_ref):
        out_ref[:, :DIM] = in_ref[...]

    return pl.pallas_call(
        body,
        grid=(v // blk,),
        in_specs=[pl.BlockSpec((blk, DIM), lambda i: (i, 0))],
        out_specs=pl.BlockSpec((blk, DIM_PAD), lambda i: (i, 0)),
        out_shape=jax.ShapeDtypeStruct((v, DIM_PAD), jnp.float32),
    )(table)


def _embed_gather(idx_grp, table, n_batch, num_cores, num_subcores):
    b_per_w = idx_grp.shape[1]

    mesh = plsc.VectorSubcoreMesh(core_axis_name="c", subcore_axis_name="s")

    nbuf = 4

    @functools.partial(
        pl.kernel,
        mesh=mesh,
        out_type=jax.ShapeDtypeStruct((n_batch * SEQ_PAD, DIM_PAD), jnp.float32),
        scratch_types=[
            pltpu.VMEM((b_per_w, SEQ_PAD), jnp.int32),
            [pltpu.VMEM((SEQ_PAD, DIM_PAD), jnp.float32) for _ in range(nbuf)],
            [pltpu.SemaphoreType.DMA for _ in range(nbuf)],
            [pltpu.SemaphoreType.DMA for _ in range(nbuf)],
        ],
    )
    def k(idx_hbm, table_hbm, out_hbm, idx_v, rows, gsems, wsems):
        wid = lax.axis_index("s") * num_cores + lax.axis_index("c")
        base_b = wid * b_per_w
        pltpu.sync_copy(idx_hbm.at[wid], idx_v)

        def g_start(jb, p):
            pltpu.async_copy(table_hbm.at[idx_v.at[jb]], rows[p], gsems[p])

        def wb_start(jb, p):
            pltpu.async_copy(
                rows[p], out_hbm.at[pl.ds((base_b + jb) * SEQ_PAD, SEQ_PAD)],
                wsems[p],
            )

        for p in range(nbuf):
            g_start(p, p)

        @pl.loop(0, b_per_w, step=nbuf)
        def _(jb):
            for p in range(nbuf):
                j = jb + p
                pltpu.make_async_copy(
                    table_hbm.at[idx_v.at[j]], rows[p], gsems[p]
                ).wait()
                wb_start(j, p)

                @pl.when(j + nbuf < b_per_w)
                def _():
                    pltpu.make_async_copy(
                        rows[p],
                        out_hbm.at[pl.ds((base_b + j) * SEQ_PAD, SEQ_PAD)],
                        wsems[p],
                    ).wait()
                    g_start(j + nbuf, p)

        for p in range(nbuf):
            j = b_per_w - nbuf + p
            pltpu.make_async_copy(
                rows[p],
                out_hbm.at[pl.ds((base_b + j) * SEQ_PAD, SEQ_PAD)],
                wsems[p],
            ).wait()

    return k(idx_grp, table)


def kernel(xc, xw, table):
    del xc
    b, l = xw.shape
    info = plsc.get_sparse_core_info()
    nw = info.num_cores * info.num_subcores
    idx = xw.reshape(nw, b // nw, l).astype(jnp.int32)
    idx = jnp.pad(idx, ((0, 0), (0, 0), (0, SEQ_PAD - SEQ)), mode="edge")
    table_p = _pad_cols_tc(table)
    out = _embed_gather(idx, table_p, b, info.num_cores, info.num_subcores)
    return out.reshape(b, SEQ_PAD, DIM_PAD)[:, :SEQ, :DIM]
